# baseline (speedup 1.0000x reference)
import functools

import jax
import jax.numpy as jnp
from jax import lax
from jax.experimental import pallas as pl
from jax.experimental.pallas import tpu as pltpu
from jax.experimental.pallas import tpu_sc as plsc

NC = 2
NS = 16
NW = NC * NS
BW = 128
DP = 128
L = 16


@functools.lru_cache(maxsize=None)
def _build(b, s, d):
    nh = s // 2

    mesh = plsc.VectorSubcoreMesh(core_axis_name="c", subcore_axis_name="s")

    @functools.partial(
        pl.kernel,
        out_type=jax.ShapeDtypeStruct((s, d, b), jnp.float32),
        mesh=mesh,
        scratch_types=[
            pltpu.VMEM((s, BW), jnp.int32),
            pltpu.VMEM((2, BW, DP), jnp.float32),
            pltpu.VMEM((2, d, BW), jnp.float32),
            pltpu.SemaphoreType.DMA,
            pltpu.SemaphoreType.DMA,
        ],
        compiler_params=pltpu.CompilerParams(
            use_tc_tiling_on_sc=True, needs_layout_passes=False),
    )
    def k(idsT_hbm, table_hbm, out_hbm, idx_v, bufA, bufT, gsem, wsem):
        wid = lax.axis_index("s") * NC + lax.axis_index("c")
        b0 = wid * BW
        pltpu.sync_copy(idsT_hbm.at[:, pl.ds(b0, BW)], idx_v)

        rows = [lax.iota(jnp.int32, L) + (g * L) for g in range(d // L)]

        def fire_gather(j, c):
            pltpu.async_copy(table_hbm.at[idx_v.at[j]], bufA.at[c], gsem)

        def drain_g():
            pltpu.make_async_copy(
                table_hbm.at[pl.ds(0, BW)], bufA.at[0], gsem).wait()

        def drain_w():
            pltpu.make_async_copy(
                out_hbm.at[0, :, pl.ds(0, BW)], bufT.at[0], wsem).wait()

        def transpose(c):
            def grp(g, carry):
                bb = g * L
                for j in range(L):
                    col = rows[0] * 0 + (bb + j)
                    for q in range(d // L):
                        v = bufA[c, bb + j, pl.ds(q * L, L)]
                        plsc.store_scatter(bufT.at[c], [rows[q], col], v)
                return carry
            lax.fori_loop(0, BW // L, grp, 0)

        def fire_write(j, c):
            pltpu.async_copy(bufT.at[c], out_hbm.at[j, :, pl.ds(b0, BW)], wsem)

        fire_gather(0, 0)

        def body(h, carry):
            for c in (0, 1):
                j = 2 * h + c

                @pl.when(j + 1 < s)
                def _():
                    fire_gather(j + 1, 1 - c)

                drain_g()

                @pl.when(h > 0)
                def _():
                    drain_w()

                transpose(c)
                fire_write(j, c)
            return carry

        lax.fori_loop(0, nh, body, 0)
        drain_w()
        drain_w()

    return k


def kernel(input_ids, table):
    b, s = input_ids.shape
    v, d = table.shape
    idsT = input_ids.astype(jnp.int32).T
    table_p = jnp.pad(table, ((0, 0), (0, DP - d)))
    out_t = _build(b, s, d)(idsT, table_p)
    return out_t.transpose(2, 0, 1)
